# baseline (speedup 1.0000x reference)
import functools

import jax
import jax.numpy as jnp
from jax import lax
from jax.experimental import pallas as pl
from jax.experimental.pallas import tpu as pltpu
from jax.experimental.pallas import tpu_sc as plsc

_E = 64
_K = 2
_H = 1024
_F = 512
_FS = 512
_C = 192
_N = 2048
_EC = _E * _C
_XR = (_E + 1) * _C
_WB = 16

_NC = 2
_NS = 16
_NW = _NC * _NS
_TPW = _N // _NW


def _sc_mesh():
    return plsc.VectorSubcoreMesh(core_axis_name="c", subcore_axis_name="s",
                                  num_cores=_NC, num_subcores=_NS)


_HI_MASK = -65536


def _rne_bf16_bits(v):
    r = lax.bitcast_convert_type(v, jnp.int32)
    r = r + 0x7FFF + (lax.shift_right_logical(r, 16) & 1)
    return r & _HI_MASK


def _pack_pair(lo_f32, hi_f32):
    lo = lax.shift_right_logical(_rne_bf16_bits(lo_f32), 16)
    hi = _rne_bf16_bits(hi_f32)
    return hi | lo


def _unpack_lo(p):
    return lax.bitcast_convert_type(lax.shift_left(p, 16), jnp.float32)


def _unpack_hi(p):
    return lax.bitcast_convert_type(p & _HI_MASK, jnp.float32)


def _router_body(x_ref, wg_ref, b_ref, loc_ref, w_ref, xp_ref):
    x = x_ref[...]
    xp_ref[...] = _pack_pair(x[:, :_H // 2], x[:, _H // 2:])
    logits = lax.dot_general(x, wg_ref[...], (((1,), (1,)), ((), ())),
                             preferred_element_type=jnp.float32)
    scores = jax.nn.sigmoid(logits)
    biased = scores + b_ref[...]
    ie = lax.broadcasted_iota(jnp.int32, (_N, _E), 1)

    m1 = jnp.max(biased, axis=1, keepdims=True)
    id1 = jnp.min(jnp.where(biased == m1, ie, _E), axis=1, keepdims=True)
    oh1 = ie == id1
    b2 = jnp.where(oh1, -jnp.inf, biased)
    m2 = jnp.max(b2, axis=1, keepdims=True)
    id2 = jnp.min(jnp.where(b2 == m2, ie, _E), axis=1, keepdims=True)
    oh2 = ie == id2

    w1 = jnp.sum(jnp.where(oh1, scores, 0.0), axis=1, keepdims=True)
    w2 = jnp.sum(jnp.where(oh2, scores, 0.0), axis=1, keepdims=True)
    s = w1 + w2

    oh_t = oh1.astype(jnp.int32) + oh2.astype(jnp.int32)
    cum = oh_t
    sft = 1
    while sft < _N:
        cum = cum + jnp.concatenate(
            [jnp.zeros((sft, _E), jnp.int32), cum[:-sft, :]], axis=0)
        sft *= 2
    pexcl = cum - oh_t
    rank0 = jnp.sum(jnp.where(oh1, pexcl, 0), axis=1)
    rank1 = jnp.sum(jnp.where(oh2, pexcl, 0), axis=1)

    id1f = jnp.min(jnp.where(oh1, ie, _E), axis=1)
    id2f = jnp.min(jnp.where(oh2, ie, _E), axis=1)
    loc0 = jnp.where(rank0 < _C, id1f * _C + rank0, _EC)
    loc1 = jnp.where(rank1 < _C, id2f * _C + rank1, _EC)

    loc_ref[0:1, :] = loc0[None, :]
    loc_ref[1:2, :] = loc1[None, :]
    w_ref[0:1, :, :] = jnp.broadcast_to(w1 / s, (_N, 16))[None]
    w_ref[1:2, :, :] = jnp.broadcast_to(w2 / s, (_N, 16))[None]


def _router(x, w_gate, bias):
    return pl.pallas_call(
        _router_body,
        out_shape=[
            jax.ShapeDtypeStruct((2, _N), jnp.int32),
            jax.ShapeDtypeStruct((2, _N, 16), jnp.float32),
            jax.ShapeDtypeStruct((_N, _H // 2), jnp.int32),
        ],
    )(x, w_gate, bias)


def _dispatch_body(x_hbm, loc_hbm, xbuf_hbm, xrows, idx0, idx1, s0, s1):
    wid = lax.axis_index("s") * _NC + lax.axis_index("c")
    base = wid * _TPW
    ci = pltpu.async_copy(x_hbm.at[pl.ds(base, _TPW)], xrows, s0)
    pltpu.sync_copy(loc_hbm.at[0, pl.ds(base, _TPW)], idx0)
    pltpu.sync_copy(loc_hbm.at[1, pl.ds(base, _TPW)], idx1)
    ci.wait()

    c0 = pltpu.async_copy(xrows, xbuf_hbm.at[idx0], s0)
    c1 = pltpu.async_copy(xrows, xbuf_hbm.at[idx1], s1)
    c0.wait()
    c1.wait()


def _dispatch(x, loc01):
    fn = pl.kernel(
        _dispatch_body,
        out_type=jax.ShapeDtypeStruct((_XR, _H // 2), jnp.int32),
        mesh=_sc_mesh(),
        scratch_types=[
            pltpu.VMEM((_TPW, _H // 2), jnp.int32),
            pltpu.VMEM((_TPW,), jnp.int32),
            pltpu.VMEM((_TPW,), jnp.int32),
            pltpu.SemaphoreType.DMA,
            pltpu.SemaphoreType.DMA,
        ],
    )
    return fn(x, loc01)


def _experts_body(x_ref, wgu_ref, wd_ref, out_ref, acc_ref):
    e = pl.program_id(0)
    f = pl.program_id(1)
    xp = x_ref[...]
    xlo = _unpack_lo(xp).astype(jnp.bfloat16)
    xhi = _unpack_hi(xp).astype(jnp.bfloat16)
    w4 = wgu_ref[0]
    wg = w4[:, 0, :]
    wu = w4[:, 1, :]
    g = (jnp.dot(xlo, wg[:_H // 2].astype(jnp.bfloat16),
                 preferred_element_type=jnp.float32)
         + jnp.dot(xhi, wg[_H // 2:].astype(jnp.bfloat16),
                   preferred_element_type=jnp.float32))
    u = (jnp.dot(xlo, wu[:_H // 2].astype(jnp.bfloat16),
                 preferred_element_type=jnp.float32)
         + jnp.dot(xhi, wu[_H // 2:].astype(jnp.bfloat16),
                   preferred_element_type=jnp.float32))
    act = (g * jax.nn.sigmoid(g) * u).astype(jnp.bfloat16)
    eo = jnp.dot(act, wd_ref[0].astype(jnp.bfloat16),
                 preferred_element_type=jnp.float32)

    @pl.when(f == 0)
    def _():
        acc_ref[...] = eo

    @pl.when(f == 1)
    def _():
        total = acc_ref[...] + eo
        packed = _pack_pair(total[:, :_H // 2], total[:, _H // 2:])
        pred = lax.broadcast(e < _E, (_C, _H // 2))
        out_ref[...] = jnp.where(pred, packed, 0)


def _experts(xbuf, w_gate_up, w_down):
    wgu4 = w_gate_up.reshape(_E, _H, 2, _F)
    return pl.pallas_call(
        _experts_body,
        grid=(_E + 1, 2),
        in_specs=[
            pl.BlockSpec((_C, _H // 2), lambda e, f: (e, 0)),
            pl.BlockSpec((1, _H, 2, _F // 2),
                         lambda e, f: (jnp.minimum(e, _E - 1), 0, 0, f)),
            pl.BlockSpec((1, _F // 2, _H),
                         lambda e, f: (jnp.minimum(e, _E - 1), f, 0)),
        ],
        out_specs=pl.BlockSpec((_C, _H // 2), lambda e, f: (e, 0)),
        out_shape=jax.ShapeDtypeStruct((_XR, _H // 2), jnp.int32),
        scratch_shapes=[pltpu.VMEM((_C, _H), jnp.float32)],
    )(xbuf, wgu4, w_down)


_FB = 256


def _finalize_body(x_ref, wgu_ref, wd_ref, g0_ref, g1_ref, w_ref, out_ref):
    xb = x_ref[...].astype(jnp.bfloat16)
    gu = jnp.dot(xb, wgu_ref[...].astype(jnp.bfloat16),
                 preferred_element_type=jnp.float32)
    g = gu[:, :_FS]
    u = gu[:, _FS:]
    act = (g * jax.nn.sigmoid(g) * u).astype(jnp.bfloat16)
    sh = jnp.dot(act, wd_ref[...].astype(jnp.bfloat16),
                 preferred_element_type=jnp.float32)
    p0 = g0_ref[0]
    p1 = g1_ref[0]
    w0 = w_ref[0, :, 0:1]
    w1 = w_ref[1, :, 0:1]
    r_lo = _unpack_lo(p0) * w0 + _unpack_lo(p1) * w1
    r_hi = _unpack_hi(p0) * w0 + _unpack_hi(p1) * w1
    out_ref[...] = jnp.concatenate([r_lo, r_hi], axis=1) + sh


def _finalize(x, ws_gate_up, ws_down, g01, w01):
    return pl.pallas_call(
        _finalize_body,
        grid=(_N // _FB,),
        in_specs=[
            pl.BlockSpec((_FB, _H), lambda i: (i, 0)),
            pl.BlockSpec((_H, 2 * _FS), lambda i: (0, 0)),
            pl.BlockSpec((_FS, _H), lambda i: (0, 0)),
            pl.BlockSpec((1, _FB, _H // 2), lambda i: (0, i, 0)),
            pl.BlockSpec((1, _FB, _H // 2), lambda i: (1, i, 0)),
            pl.BlockSpec((2, _FB, 16), lambda i: (0, i, 0)),
        ],
        out_specs=pl.BlockSpec((_FB, _H), lambda i: (i, 0)),
        out_shape=jax.ShapeDtypeStruct((_N, _H), jnp.float32),
    )(x, ws_gate_up, ws_down, g01, g01, w01)


def _combine_body(eo_hbm, loc_hbm, g_hbm, p0, p1, idx0, idx1, s0, s1):
    wid = lax.axis_index("s") * _NC + lax.axis_index("c")
    base = wid * _TPW
    pltpu.sync_copy(loc_hbm.at[0, pl.ds(base, _TPW)], idx0)
    pltpu.sync_copy(loc_hbm.at[1, pl.ds(base, _TPW)], idx1)
    ca = pltpu.async_copy(eo_hbm.at[idx0], p0, s0)
    cb = pltpu.async_copy(eo_hbm.at[idx1], p1, s1)
    ca.wait()
    cb.wait()
    cw0 = pltpu.async_copy(p0, g_hbm.at[0, pl.ds(base, _TPW)], s0)
    cw1 = pltpu.async_copy(p1, g_hbm.at[1, pl.ds(base, _TPW)], s1)
    cw0.wait()
    cw1.wait()


def _combine(eo, loc01):
    fn = pl.kernel(
        _combine_body,
        out_type=jax.ShapeDtypeStruct((2, _N, _H // 2), jnp.int32),
        mesh=_sc_mesh(),
        scratch_types=[
            pltpu.VMEM((_TPW, _H // 2), jnp.int32),
            pltpu.VMEM((_TPW, _H // 2), jnp.int32),
            pltpu.VMEM((_TPW,), jnp.int32),
            pltpu.VMEM((_TPW,), jnp.int32),
            pltpu.SemaphoreType.DMA,
            pltpu.SemaphoreType.DMA,
        ],
    )
    return fn(eo, loc01)


def kernel(hidden_states, W_gate, expert_bias, W_gate_up, W_down,
           Ws_gate_up, Ws_down):
    orig_shape = hidden_states.shape
    x = hidden_states.reshape(-1, _H)
    loc01, w01, xp = _router(x, W_gate, expert_bias.reshape(1, _E))
    xbuf = _dispatch(xp, loc01)
    eo = _experts(xbuf, W_gate_up, W_down)
    g01 = _combine(eo, loc01)
    out = _finalize(x, Ws_gate_up, Ws_down, g01, w01)
    return out.reshape(orig_shape)


# speedup vs baseline: 3.6825x; 3.6825x over previous
import functools

import jax
import jax.numpy as jnp
from jax import lax
from jax.experimental import pallas as pl
from jax.experimental.pallas import tpu as pltpu
from jax.experimental.pallas import tpu_sc as plsc

_E = 64
_K = 2
_H = 1024
_F = 512
_FS = 512
_C = 192
_N = 2048
_EC = _E * _C
_XR = (_E + 1) * _C
_WB = 16

_NC = 2
_NS = 16
_NW = _NC * _NS
_TPW = _N // _NW


def _sc_mesh():
    return plsc.VectorSubcoreMesh(core_axis_name="c", subcore_axis_name="s",
                                  num_cores=_NC, num_subcores=_NS)


_HI_MASK = -65536


def _rne_bf16_bits(v):
    r = lax.bitcast_convert_type(v, jnp.int32)
    r = r + 0x7FFF + (lax.shift_right_logical(r, 16) & 1)
    return r & _HI_MASK


def _pack_pair(lo_f32, hi_f32):
    lo = lax.shift_right_logical(_rne_bf16_bits(lo_f32), 16)
    hi = _rne_bf16_bits(hi_f32)
    return hi | lo


def _unpack_lo(p):
    return lax.bitcast_convert_type(lax.shift_left(p, 16), jnp.float32)


def _unpack_hi(p):
    return lax.bitcast_convert_type(p & _HI_MASK, jnp.float32)


def _router_body(x_ref, wg_ref, b_ref, loc_ref, w_ref, xp_ref):
    x = x_ref[...]
    xp_ref[...] = _pack_pair(x[:, :_H // 2], x[:, _H // 2:])
    logits = lax.dot_general(x, wg_ref[...], (((1,), (1,)), ((), ())),
                             preferred_element_type=jnp.float32)
    scores = jax.nn.sigmoid(logits)
    biased = scores + b_ref[...]
    ie = lax.broadcasted_iota(jnp.int32, (_N, _E), 1)

    m1 = jnp.max(biased, axis=1, keepdims=True)
    id1 = jnp.min(jnp.where(biased == m1, ie, _E), axis=1, keepdims=True)
    oh1 = ie == id1
    b2 = jnp.where(oh1, -jnp.inf, biased)
    m2 = jnp.max(b2, axis=1, keepdims=True)
    id2 = jnp.min(jnp.where(b2 == m2, ie, _E), axis=1, keepdims=True)
    oh2 = ie == id2

    w1 = jnp.sum(jnp.where(oh1, scores, 0.0), axis=1, keepdims=True)
    w2 = jnp.sum(jnp.where(oh2, scores, 0.0), axis=1, keepdims=True)
    s = w1 + w2

    oh_t = oh1.astype(jnp.int32) + oh2.astype(jnp.int32)
    cum = oh_t
    sft = 1
    while sft < _N:
        cum = cum + jnp.concatenate(
            [jnp.zeros((sft, _E), jnp.int32), cum[:-sft, :]], axis=0)
        sft *= 2
    pexcl = cum - oh_t
    rank0 = jnp.sum(jnp.where(oh1, pexcl, 0), axis=1)
    rank1 = jnp.sum(jnp.where(oh2, pexcl, 0), axis=1)

    id1f = jnp.min(jnp.where(oh1, ie, _E), axis=1)
    id2f = jnp.min(jnp.where(oh2, ie, _E), axis=1)
    loc0 = jnp.where(rank0 < _C, id1f * _C + rank0, _EC)
    loc1 = jnp.where(rank1 < _C, id2f * _C + rank1, _EC)

    loc_ref[0:1, :] = loc0[None, :]
    loc_ref[1:2, :] = loc1[None, :]
    w_ref[0:1, :, :] = jnp.broadcast_to(w1 / s, (_N, 16))[None]
    w_ref[1:2, :, :] = jnp.broadcast_to(w2 / s, (_N, 16))[None]


def _router(x, w_gate, bias):
    return pl.pallas_call(
        _router_body,
        out_shape=[
            jax.ShapeDtypeStruct((2, _N), jnp.int32),
            jax.ShapeDtypeStruct((2, _N, 16), jnp.float32),
            jax.ShapeDtypeStruct((_N, _H // 2), jnp.int32),
        ],
    )(x, w_gate, bias)


def _dispatch_body(x_hbm, loc_hbm, xbuf_hbm, xrows, idx0, idx1, s0, s1):
    wid = lax.axis_index("s") * _NC + lax.axis_index("c")
    base = wid * _TPW
    ci = pltpu.async_copy(x_hbm.at[pl.ds(base, _TPW)], xrows, s0)
    pltpu.sync_copy(loc_hbm.at[0, pl.ds(base, _TPW)], idx0)
    pltpu.sync_copy(loc_hbm.at[1, pl.ds(base, _TPW)], idx1)
    ci.wait()

    c0 = pltpu.async_copy(xrows, xbuf_hbm.at[idx0], s0)
    c1 = pltpu.async_copy(xrows, xbuf_hbm.at[idx1], s1)
    c0.wait()
    c1.wait()


def _dispatch(x, loc01):
    fn = pl.kernel(
        _dispatch_body,
        out_type=jax.ShapeDtypeStruct((_XR, _H // 2), jnp.int32),
        mesh=_sc_mesh(),
        scratch_types=[
            pltpu.VMEM((_TPW, _H // 2), jnp.int32),
            pltpu.VMEM((_TPW,), jnp.int32),
            pltpu.VMEM((_TPW,), jnp.int32),
            pltpu.SemaphoreType.DMA,
            pltpu.SemaphoreType.DMA,
        ],
    )
    return fn(x, loc01)


def _experts_body(x_ref, wgu_ref, wd_ref, out_ref):
    e = pl.program_id(0)
    xp = x_ref[...]
    xlo = _unpack_lo(xp).astype(jnp.bfloat16)
    xhi = _unpack_hi(xp).astype(jnp.bfloat16)
    wgu = wgu_ref[0]
    wlo = wgu[:_H // 2, :].astype(jnp.bfloat16)
    whi = wgu[_H // 2:, :].astype(jnp.bfloat16)
    gu = (jnp.dot(xlo, wlo, preferred_element_type=jnp.float32)
          + jnp.dot(xhi, whi, preferred_element_type=jnp.float32))
    g = gu[:, :_F]
    u = gu[:, _F:]
    act = (g * jax.nn.sigmoid(g) * u).astype(jnp.bfloat16)
    eo = jnp.dot(act, wd_ref[0].astype(jnp.bfloat16),
                 preferred_element_type=jnp.float32)
    packed = _pack_pair(eo[:, :_H // 2], eo[:, _H // 2:])
    pred = lax.broadcast(e < _E, (_C, _H // 2))
    out_ref[...] = jnp.where(pred, packed, 0)


def _experts(xbuf, w_gate_up, w_down):
    return pl.pallas_call(
        _experts_body,
        grid=(_E + 1,),
        in_specs=[
            pl.BlockSpec((_C, _H // 2), lambda e: (e, 0)),
            pl.BlockSpec((1, _H, 2 * _F), lambda e: (jnp.minimum(e, _E - 1), 0, 0)),
            pl.BlockSpec((1, _F, _H), lambda e: (jnp.minimum(e, _E - 1), 0, 0)),
        ],
        out_specs=pl.BlockSpec((_C, _H // 2), lambda e: (e, 0)),
        out_shape=jax.ShapeDtypeStruct((_XR, _H // 2), jnp.int32),
    )(xbuf, w_gate_up, w_down)


_FB = 256


def _finalize_body(x_ref, wgu_ref, wd_ref, g0_ref, g1_ref, w_ref, out_ref):
    xp = x_ref[...]
    xlo = _unpack_lo(xp).astype(jnp.bfloat16)
    xhi = _unpack_hi(xp).astype(jnp.bfloat16)
    wsgu = wgu_ref[...]
    gu = (jnp.dot(xlo, wsgu[:_H // 2].astype(jnp.bfloat16),
                  preferred_element_type=jnp.float32)
          + jnp.dot(xhi, wsgu[_H // 2:].astype(jnp.bfloat16),
                    preferred_element_type=jnp.float32))
    g = gu[:, :_FS]
    u = gu[:, _FS:]
    act = (g * jax.nn.sigmoid(g) * u).astype(jnp.bfloat16)
    sh = jnp.dot(act, wd_ref[...].astype(jnp.bfloat16),
                 preferred_element_type=jnp.float32)
    p0 = g0_ref[0]
    p1 = g1_ref[0]
    w0 = w_ref[0, :, 0:1]
    w1 = w_ref[1, :, 0:1]
    r_lo = _unpack_lo(p0) * w0 + _unpack_lo(p1) * w1
    r_hi = _unpack_hi(p0) * w0 + _unpack_hi(p1) * w1
    out_ref[...] = jnp.concatenate([r_lo, r_hi], axis=1) + sh


def _finalize(xp, ws_gate_up, ws_down, g01, w01):
    return pl.pallas_call(
        _finalize_body,
        grid=(_N // _FB,),
        in_specs=[
            pl.BlockSpec((_FB, _H // 2), lambda i: (i, 0)),
            pl.BlockSpec((_H, 2 * _FS), lambda i: (0, 0)),
            pl.BlockSpec((_FS, _H), lambda i: (0, 0)),
            pl.BlockSpec((1, _FB, _H // 2), lambda i: (0, i, 0)),
            pl.BlockSpec((1, _FB, _H // 2), lambda i: (1, i, 0)),
            pl.BlockSpec((2, _FB, 16), lambda i: (0, i, 0)),
        ],
        out_specs=pl.BlockSpec((_FB, _H), lambda i: (i, 0)),
        out_shape=jax.ShapeDtypeStruct((_N, _H), jnp.float32),
    )(xp, ws_gate_up, ws_down, g01, g01, w01)


def _combine_body(eo_hbm, loc_hbm, g_hbm, p0, p1, idx0, idx1, s0, s1):
    wid = lax.axis_index("s") * _NC + lax.axis_index("c")
    base = wid * _TPW
    pltpu.sync_copy(loc_hbm.at[0, pl.ds(base, _TPW)], idx0)
    pltpu.sync_copy(loc_hbm.at[1, pl.ds(base, _TPW)], idx1)
    ca = pltpu.async_copy(eo_hbm.at[idx0], p0, s0)
    cb = pltpu.async_copy(eo_hbm.at[idx1], p1, s1)
    ca.wait()
    cb.wait()
    cw0 = pltpu.async_copy(p0, g_hbm.at[0, pl.ds(base, _TPW)], s0)
    cw1 = pltpu.async_copy(p1, g_hbm.at[1, pl.ds(base, _TPW)], s1)
    cw0.wait()
    cw1.wait()


def _combine(eo, loc01):
    fn = pl.kernel(
        _combine_body,
        out_type=jax.ShapeDtypeStruct((2, _N, _H // 2), jnp.int32),
        mesh=_sc_mesh(),
        scratch_types=[
            pltpu.VMEM((_TPW, _H // 2), jnp.int32),
            pltpu.VMEM((_TPW, _H // 2), jnp.int32),
            pltpu.VMEM((_TPW,), jnp.int32),
            pltpu.VMEM((_TPW,), jnp.int32),
            pltpu.SemaphoreType.DMA,
            pltpu.SemaphoreType.DMA,
        ],
    )
    return fn(eo, loc01)


def kernel(hidden_states, W_gate, expert_bias, W_gate_up, W_down,
           Ws_gate_up, Ws_down):
    orig_shape = hidden_states.shape
    x = hidden_states.reshape(-1, _H)
    loc01, w01, xp = _router(x, W_gate, expert_bias.reshape(1, _E))
    xbuf = _dispatch(xp, loc01)
    eo = _experts(xbuf, W_gate_up, W_down)
    g01 = _combine(eo, loc01)
    out = _finalize(xp, Ws_gate_up, Ws_down, g01, w01)
    return out.reshape(orig_shape)
